# baseline (speedup 1.0000x reference)
import functools

import jax
import jax.numpy as jnp
from jax import lax
from jax.experimental import pallas as pl
from jax.experimental.pallas import tpu as pltpu
from jax.experimental.pallas import tpu_sc as plsc


def kernel(etypes, table):
    E = etypes.shape[0]
    V, D = table.shape

    info = plsc.get_sparse_core_info()
    NC, NS = info.num_cores, info.num_subcores
    NW = NC * NS

    per_w = E // NW
    assert per_w * NW == E and per_w % 8 == 0

    CH = 128
    n_full = per_w // CH
    rem = per_w - n_full * CH

    mesh = plsc.VectorSubcoreMesh(core_axis_name="c", subcore_axis_name="s")

    @functools.partial(
        pl.kernel,
        mesh=mesh,
        compiler_params=pltpu.CompilerParams(use_tc_tiling_on_sc=False),
        out_type=jax.ShapeDtypeStruct((E, D), jnp.float32),
        scratch_types=[
            pltpu.VMEM((per_w,), jnp.int32),
            pltpu.VMEM((CH, D), jnp.float32),
            pltpu.SemaphoreType.DMA,
        ],
    )
    def emb_kernel(etypes_hbm, table_hbm, out_hbm, idx_v, rows_v, sem):
        wid = lax.axis_index("s") * NC + lax.axis_index("c")
        base = wid * per_w
        pltpu.sync_copy(etypes_hbm.at[pl.ds(base, per_w)], idx_v)

        def body(k, carry):
            off = k * CH
            pltpu.async_copy(
                table_hbm.at[idx_v.at[pl.ds(off, CH)]], rows_v, sem
            ).wait()
            pltpu.sync_copy(rows_v, out_hbm.at[pl.ds(base + off, CH)])
            return carry

        lax.fori_loop(0, n_full, body, 0)

        if rem:
            off = n_full * CH
            pltpu.async_copy(
                table_hbm.at[idx_v.at[pl.ds(off, rem)]],
                rows_v.at[pl.ds(0, rem)],
                sem,
            ).wait()
            pltpu.sync_copy(
                rows_v.at[pl.ds(0, rem)], out_hbm.at[pl.ds(base + off, rem)]
            )

    return emb_kernel(etypes, table)


# speedup vs baseline: 4.2952x; 4.2952x over previous
import functools

import jax
import jax.numpy as jnp
from jax import lax
from jax.experimental import pallas as pl
from jax.experimental.pallas import tpu as pltpu
from jax.experimental.pallas import tpu_sc as plsc


def kernel(etypes, table):
    E = etypes.shape[0]
    V, D = table.shape

    info = plsc.get_sparse_core_info()
    NC, NS = info.num_cores, info.num_subcores
    NW = NC * NS

    per_w = E // NW
    assert per_w * NW == E and per_w % 8 == 0

    CH = 128
    n_full = per_w // CH
    rem = per_w - n_full * CH

    mesh = plsc.VectorSubcoreMesh(core_axis_name="c", subcore_axis_name="s")

    @functools.partial(
        pl.kernel,
        mesh=mesh,
        compiler_params=pltpu.CompilerParams(use_tc_tiling_on_sc=False),
        out_type=jax.ShapeDtypeStruct((E, D), jnp.float32),
        scratch_types=[
            pltpu.VMEM((per_w,), jnp.int32),
            pltpu.VMEM((CH, D), jnp.float32),
            pltpu.VMEM((V, D), jnp.float32),
            pltpu.VMEM_SHARED((V, D), jnp.float32),
            pltpu.SemaphoreType.DMA,
        ],
    )
    def emb_kernel(
        etypes_hbm, table_hbm, out_hbm, idx_v, rows_v, tab_v, tab_sp, sem
    ):
        sid = lax.axis_index("s")
        wid = sid * NC + lax.axis_index("c")
        base = wid * per_w

        @pl.when(sid == 0)
        def _stage():
            pltpu.sync_copy(table_hbm, tab_v)
            pltpu.sync_copy(tab_v, tab_sp)

        pltpu.sync_copy(etypes_hbm.at[pl.ds(base, per_w)], idx_v)
        plsc.subcore_barrier()

        def body(k, carry):
            off = k * CH
            pltpu.async_copy(
                tab_sp.at[idx_v.at[pl.ds(off, CH)]], rows_v, sem
            ).wait()
            pltpu.sync_copy(rows_v, out_hbm.at[pl.ds(base + off, CH)])
            return carry

        lax.fori_loop(0, n_full, body, 0)

        if rem:
            off = n_full * CH
            pltpu.async_copy(
                tab_sp.at[idx_v.at[pl.ds(off, rem)]],
                rows_v.at[pl.ds(0, rem)],
                sem,
            ).wait()
            pltpu.sync_copy(
                rows_v.at[pl.ds(0, rem)], out_hbm.at[pl.ds(base + off, rem)]
            )

    return emb_kernel(etypes, table)


# speedup vs baseline: 5.5359x; 1.2889x over previous
import functools

import jax
import jax.numpy as jnp
from jax import lax
from jax.experimental import pallas as pl
from jax.experimental.pallas import tpu as pltpu
from jax.experimental.pallas import tpu_sc as plsc


def kernel(etypes, table):
    E = etypes.shape[0]
    V, D = table.shape

    info = plsc.get_sparse_core_info()
    NC, NS = info.num_cores, info.num_subcores
    NW = NC * NS

    per_w = E // NW
    assert per_w * NW == E and per_w % 8 == 0

    CH = 128
    n_full = per_w // CH
    rem = per_w - n_full * CH

    mesh = plsc.VectorSubcoreMesh(core_axis_name="c", subcore_axis_name="s")

    @functools.partial(
        pl.kernel,
        mesh=mesh,
        compiler_params=pltpu.CompilerParams(use_tc_tiling_on_sc=True),
        out_type=jax.ShapeDtypeStruct((E, D), jnp.float32),
        scratch_types=[
            pltpu.VMEM((per_w,), jnp.int32),
            pltpu.VMEM((CH, D), jnp.float32),
            pltpu.VMEM((V, D), jnp.float32),
            pltpu.VMEM_SHARED((V, D), jnp.float32),
            pltpu.SemaphoreType.DMA,
        ],
    )
    def emb_kernel(
        etypes_hbm, table_hbm, out_hbm, idx_v, rows_v, tab_v, tab_sp, sem
    ):
        sid = lax.axis_index("s")
        wid = sid * NC + lax.axis_index("c")
        base = wid * per_w

        @pl.when(sid == 0)
        def _stage():
            pltpu.sync_copy(table_hbm, tab_v)
            pltpu.sync_copy(tab_v, tab_sp)

        pltpu.sync_copy(etypes_hbm.at[pl.ds(base, per_w)], idx_v)
        plsc.subcore_barrier()

        def body(k, carry):
            off = k * CH
            pltpu.async_copy(
                tab_sp.at[idx_v.at[pl.ds(off, CH)]], rows_v, sem
            ).wait()
            pltpu.sync_copy(rows_v, out_hbm.at[pl.ds(base + off, CH)])
            return carry

        lax.fori_loop(0, n_full, body, 0)

        if rem:
            off = n_full * CH
            pltpu.async_copy(
                tab_sp.at[idx_v.at[pl.ds(off, rem)]],
                rows_v.at[pl.ds(0, rem)],
                sem,
            ).wait()
            pltpu.sync_copy(
                rows_v.at[pl.ds(0, rem)], out_hbm.at[pl.ds(base + off, rem)]
            )

    return emb_kernel(etypes, table)
